# baseline (speedup 1.0000x reference)
import jax
import jax.numpy as jnp
from jax.experimental import pallas as pl
from jax.experimental.pallas import tpu as pltpu

_TOP_K = 50


def _matvec_kernel(q_ref, w_ref, o_ref):
    o_ref[...] = jax.lax.dot_general(
        q_ref[...], w_ref[...], (((1,), (0,)), ((), ())),
        preferred_element_type=jnp.float32,
    )


def _loss_kernel(sim_ref, lab_ref, gl_ref, o_ref, s_ref):
    sim = sim_ref[...]
    mask = lab_ref[...] == gl_ref[0, 0]
    pos_cnt = jnp.sum(mask.astype(jnp.float32))
    pos_sum = jnp.sum(jnp.where(mask, sim, 0.0))
    pos_max = jnp.max(jnp.where(mask, sim, -jnp.inf))
    neg = jnp.where(mask, -jnp.inf, sim)
    neg_max = jnp.max(neg)
    m_all = jnp.maximum(pos_max, neg_max)
    pos_es = jnp.sum(jnp.where(mask, jnp.exp(sim - m_all), 0.0))

    s_ref[...] = neg

    def body(_, carry):
        taken, es = carry
        cur = s_ref[...]
        m = jnp.max(cur)
        eqm = cur == m
        cnt = jnp.sum(eqm.astype(jnp.float32))
        take = jnp.clip(jnp.minimum(cnt, _TOP_K - taken), 0.0, None)
        s_ref[...] = jnp.where(eqm, -jnp.inf, cur)
        return taken + cnt, es + take * jnp.exp(m - m_all)

    _, neg_es = jax.lax.fori_loop(0, _TOP_K, body, (0.0, 0.0))

    exp_sum = pos_es + neg_es
    lossv = -(pos_sum / jnp.maximum(pos_cnt, 1.0)) + m_all + jnp.log(exp_sum)
    o_ref[...] = jnp.where(pos_cnt == 0.0, 0.0, lossv)[None, None]


def kernel(gallery_feat, query_feats, gallery_label, query_labels):
    n, d = query_feats.shape
    w = gallery_feat.reshape(d, 1)

    br = 8000
    nb = n // br
    sim = pl.pallas_call(
        _matvec_kernel,
        grid=(nb,),
        in_specs=[
            pl.BlockSpec((br, d), lambda i: (i, 0)),
            pl.BlockSpec((d, 1), lambda i: (0, 0)),
        ],
        out_specs=pl.BlockSpec((br, 1), lambda i: (i, 0)),
        out_shape=jax.ShapeDtypeStruct((n, 1), jnp.float32),
    )(query_feats, w)

    rows, cols = 125, 8000
    sim2 = sim.reshape(rows, cols)
    lab2 = query_labels.reshape(rows, cols)
    gl = gallery_label.reshape(1, 1)

    loss = pl.pallas_call(
        _loss_kernel,
        out_shape=jax.ShapeDtypeStruct((1, 1), jnp.float32),
        scratch_shapes=[pltpu.VMEM((rows, cols), jnp.float32)],
    )(sim2, lab2, gl)
    return loss[0, 0]


# speedup vs baseline: 1.4931x; 1.4931x over previous
import jax
import jax.numpy as jnp
from jax.experimental import pallas as pl
from jax.experimental.pallas import tpu as pltpu

_TOP_K = 50


def _matvec_kernel(q_ref, w_ref, o_ref):
    chunks = [
        jax.lax.dot_general(
            w_ref[...], q_ref[5000 * j:5000 * (j + 1), :],
            (((1,), (1,)), ((), ())),
            preferred_element_type=jnp.float32,
        )
        for j in range(8)
    ]
    o_ref[...] = jnp.concatenate(chunks, axis=0)[None]


def _loss_kernel(sim_ref, lab_ref, gl_ref, o_ref, s_ref):
    sim = sim_ref[...]
    mask = lab_ref[...] == gl_ref[0, 0]
    pos_cnt = jnp.sum(mask.astype(jnp.float32))
    pos_sum = jnp.sum(jnp.where(mask, sim, 0.0))
    pos_max = jnp.max(jnp.where(mask, sim, -jnp.inf))
    neg = jnp.where(mask, -jnp.inf, sim)
    neg_max = jnp.max(neg)
    m_all = jnp.maximum(pos_max, neg_max)
    pos_es = jnp.sum(jnp.where(mask, jnp.exp(sim - m_all), 0.0))

    s_ref[...] = neg

    def body(_, carry):
        taken, es = carry
        cur = s_ref[...]
        m = jnp.max(cur)
        eqm = cur == m
        cnt = jnp.sum(eqm.astype(jnp.float32))
        take = jnp.clip(jnp.minimum(cnt, _TOP_K - taken), 0.0, None)
        s_ref[...] = jnp.where(eqm, -jnp.inf, cur)
        return taken + cnt, es + take * jnp.exp(m - m_all)

    _, neg_es = jax.lax.fori_loop(0, _TOP_K, body, (0.0, 0.0))

    exp_sum = pos_es + neg_es
    lossv = -(pos_sum / jnp.maximum(pos_cnt, 1.0)) + m_all + jnp.log(exp_sum)
    o_ref[...] = jnp.where(pos_cnt == 0.0, 0.0, lossv)[None, None]


def kernel(gallery_feat, query_feats, gallery_label, query_labels):
    n, d = query_feats.shape
    w = gallery_feat.reshape(1, d)

    br = 40000
    nb = n // br
    sim = pl.pallas_call(
        _matvec_kernel,
        grid=(nb,),
        in_specs=[
            pl.BlockSpec((br, d), lambda i: (i, 0)),
            pl.BlockSpec((1, d), lambda i: (0, 0)),
        ],
        out_specs=pl.BlockSpec((1, 8, 5000), lambda i: (i, 0, 0)),
        out_shape=jax.ShapeDtypeStruct((nb, 8, 5000), jnp.float32),
    )(query_feats, w)

    rows, cols = 125, 8000
    sim2 = sim.reshape(rows, cols)
    lab2 = query_labels.reshape(rows, cols)
    gl = gallery_label.reshape(1, 1)

    loss = pl.pallas_call(
        _loss_kernel,
        out_shape=jax.ShapeDtypeStruct((1, 1), jnp.float32),
        scratch_shapes=[pltpu.VMEM((rows, cols), jnp.float32)],
    )(sim2, lab2, gl)
    return loss[0, 0]


# speedup vs baseline: 1.4967x; 1.0024x over previous
import jax
import jax.numpy as jnp
from jax.experimental import pallas as pl
from jax.experimental.pallas import tpu as pltpu

_TOP_K = 50


def _matvec_kernel(q_ref, w_ref, o_ref):
    chunks = [
        jax.lax.dot_general(
            w_ref[...], q_ref[2500 * j:2500 * (j + 1), :],
            (((1,), (1,)), ((), ())),
            preferred_element_type=jnp.float32,
        )
        for j in range(8)
    ]
    o_ref[...] = jnp.concatenate(chunks, axis=0)[None]


def _loss_kernel(sim_ref, lab_ref, gl_ref, o_ref, s_ref):
    sim = sim_ref[...]
    mask = lab_ref[...] == gl_ref[0, 0]
    pos_cnt = jnp.sum(mask.astype(jnp.float32))
    pos_sum = jnp.sum(jnp.where(mask, sim, 0.0))
    pos_max = jnp.max(jnp.where(mask, sim, -jnp.inf))
    neg = jnp.where(mask, -jnp.inf, sim)
    neg_max = jnp.max(neg)
    m_all = jnp.maximum(pos_max, neg_max)
    pos_es = jnp.sum(jnp.where(mask, jnp.exp(sim - m_all), 0.0))

    s_ref[...] = neg

    def body(_, carry):
        taken, es = carry
        cur = s_ref[...]
        m = jnp.max(cur)
        eqm = cur == m
        cnt = jnp.sum(eqm.astype(jnp.float32))
        take = jnp.clip(jnp.minimum(cnt, _TOP_K - taken), 0.0, None)
        s_ref[...] = jnp.where(eqm, -jnp.inf, cur)
        return taken + cnt, es + take * jnp.exp(m - m_all)

    _, neg_es = jax.lax.fori_loop(0, _TOP_K, body, (0.0, 0.0))

    exp_sum = pos_es + neg_es
    lossv = -(pos_sum / jnp.maximum(pos_cnt, 1.0)) + m_all + jnp.log(exp_sum)
    o_ref[...] = jnp.where(pos_cnt == 0.0, 0.0, lossv)[None, None]


def kernel(gallery_feat, query_feats, gallery_label, query_labels):
    n, d = query_feats.shape
    w = gallery_feat.reshape(1, d)

    br = 20000
    nb = n // br
    sim = pl.pallas_call(
        _matvec_kernel,
        grid=(nb,),
        in_specs=[
            pl.BlockSpec((br, d), lambda i: (i, 0)),
            pl.BlockSpec((1, d), lambda i: (0, 0)),
        ],
        out_specs=pl.BlockSpec((1, 8, 2500), lambda i: (i, 0, 0)),
        out_shape=jax.ShapeDtypeStruct((nb, 8, 2500), jnp.float32),
    )(query_feats, w)

    rows, cols = 125, 8000
    sim2 = sim.reshape(rows, cols)
    lab2 = query_labels.reshape(rows, cols)
    gl = gallery_label.reshape(1, 1)

    loss = pl.pallas_call(
        _loss_kernel,
        out_shape=jax.ShapeDtypeStruct((1, 1), jnp.float32),
        scratch_shapes=[pltpu.VMEM((rows, cols), jnp.float32)],
    )(sim2, lab2, gl)
    return loss[0, 0]


# speedup vs baseline: 1.5144x; 1.0118x over previous
import jax
import jax.numpy as jnp
from jax.experimental import pallas as pl
from jax.experimental.pallas import tpu as pltpu

_TOP_K = 50
_BR = 20000
_CH = _BR // 8


def _fused_kernel(q_ref, w_ref, lab_ref, gl_ref, o_ref, s_ref):
    i = pl.program_id(0)
    nb = pl.num_programs(0)

    chunks = [
        jax.lax.dot_general(
            w_ref[...], q_ref[_CH * j:_CH * (j + 1), :],
            (((1,), (1,)), ((), ())),
            preferred_element_type=jnp.float32,
        )
        for j in range(8)
    ]
    s_ref[pl.ds(8 * i, 8), :] = jnp.concatenate(chunks, axis=0)

    @pl.when(i == nb - 1)
    def _finalize():
        sim = s_ref[...]
        mask = lab_ref[...] == gl_ref[0, 0]
        pos_cnt = jnp.sum(mask.astype(jnp.float32))
        pos_sum = jnp.sum(jnp.where(mask, sim, 0.0))
        pos_max = jnp.max(jnp.where(mask, sim, -jnp.inf))
        neg = jnp.where(mask, -jnp.inf, sim)
        neg_max = jnp.max(neg)
        m_all = jnp.maximum(pos_max, neg_max)
        pos_es = jnp.sum(jnp.where(mask, jnp.exp(sim - m_all), 0.0))

        s_ref[...] = neg

        def body(_, carry):
            taken, es = carry
            cur = s_ref[...]
            m = jnp.max(cur)
            eqm = cur == m
            cnt = jnp.sum(eqm.astype(jnp.float32))
            take = jnp.clip(jnp.minimum(cnt, _TOP_K - taken), 0.0, None)
            s_ref[...] = jnp.where(eqm, -jnp.inf, cur)
            return taken + cnt, es + take * jnp.exp(m - m_all)

        _, neg_es = jax.lax.fori_loop(0, _TOP_K, body, (0.0, 0.0))

        exp_sum = pos_es + neg_es
        lossv = (-(pos_sum / jnp.maximum(pos_cnt, 1.0))
                 + m_all + jnp.log(exp_sum))
        o_ref[...] = jnp.where(pos_cnt == 0.0, 0.0, lossv)[None, None]


def kernel(gallery_feat, query_feats, gallery_label, query_labels):
    n, d = query_feats.shape
    w = gallery_feat.reshape(1, d)
    nb = n // _BR
    rows = 8 * nb
    lab2 = query_labels.reshape(rows, _CH)
    gl = gallery_label.reshape(1, 1)

    loss = pl.pallas_call(
        _fused_kernel,
        grid=(nb,),
        in_specs=[
            pl.BlockSpec((_BR, d), lambda i: (i, 0)),
            pl.BlockSpec((1, d), lambda i: (0, 0)),
            pl.BlockSpec((rows, _CH), lambda i: (0, 0)),
            pl.BlockSpec((1, 1), lambda i: (0, 0)),
        ],
        out_specs=pl.BlockSpec((1, 1), lambda i: (0, 0)),
        out_shape=jax.ShapeDtypeStruct((1, 1), jnp.float32),
        scratch_shapes=[pltpu.VMEM((rows, _CH), jnp.float32)],
    )(query_feats, w, lab2, gl)
    return loss[0, 0]
